# baseline (speedup 1.0000x reference)
import jax
import jax.numpy as jnp
from jax.experimental import pallas as pl
from jax.experimental.pallas import tpu as pltpu
from functools import partial

_N_TOKENS = 8192
_DIM = 4096
_E = 64
_EPS = 1e-06
_TILE = 1024


def _gate_kernel(x_ref, wt_ref, b_ref, out_ref, colsum_ref, *, n_tiles, tile,
                 capacity):
    i = pl.program_id(0)

    logits = jnp.dot(x_ref[...], wt_ref[...],
                     preferred_element_type=jnp.float32) + b_ref[...]
    m = jnp.max(logits, axis=1, keepdims=True)
    s = jnp.sum(jnp.exp(logits - m), axis=1, keepdims=True)
    iota = jax.lax.broadcasted_iota(jnp.int32, logits.shape, 1)
    is_max = logits >= m
    first = jnp.min(jnp.where(is_max, iota, _E), axis=1, keepdims=True)
    masked = jnp.where(iota == first, 1.0 / s, 0.0)

    @pl.when(i == 0)
    def _init():
        colsum_ref[...] = jnp.zeros_like(colsum_ref)

    colsum_ref[...] += jnp.sum(masked, axis=0, keepdims=True)
    out_ref[pl.ds(i * tile, tile), :] = masked

    @pl.when(i == n_tiles - 1)
    def _normalize():
        scale = capacity / (colsum_ref[...] + _EPS)
        out_ref[...] = out_ref[...] * scale


def kernel(x, W, b):
    n_tiles = _N_TOKENS // _TILE
    wt = W.T
    b2 = b.reshape(1, _E)
    capacity = float(_N_TOKENS)
    return pl.pallas_call(
        partial(_gate_kernel, n_tiles=n_tiles, tile=_TILE, capacity=capacity),
        grid=(n_tiles,),
        in_specs=[
            pl.BlockSpec((_TILE, _DIM), lambda i: (i, 0)),
            pl.BlockSpec((_DIM, _E), lambda i: (0, 0)),
            pl.BlockSpec((1, _E), lambda i: (0, 0)),
        ],
        out_specs=pl.BlockSpec((_N_TOKENS, _E), lambda i: (0, 0)),
        out_shape=jax.ShapeDtypeStruct((_N_TOKENS, _E), jnp.float32),
        scratch_shapes=[pltpu.VMEM((1, _E), jnp.float32)],
    )(x, wt, b2)


# speedup vs baseline: 1.0715x; 1.0715x over previous
import jax
import jax.numpy as jnp
from jax.experimental import pallas as pl
from jax.experimental.pallas import tpu as pltpu
from functools import partial

_N_TOKENS = 8192
_DIM = 4096
_E = 64
_EPS = 1e-06
_TILE = 1024


def _gate_kernel(x_ref, wt_ref, b_ref, out_ref, colsum_ref, *, n_tiles, tile,
                 capacity):
    i = pl.program_id(0)

    logits = jax.lax.dot_general(
        x_ref[...], wt_ref[...],
        dimension_numbers=(((1,), (1,)), ((), ())),
        preferred_element_type=jnp.float32) + b_ref[...]
    m = jnp.max(logits, axis=1, keepdims=True)
    s = jnp.sum(jnp.exp(logits - m), axis=1, keepdims=True)
    iota = jax.lax.broadcasted_iota(jnp.int32, logits.shape, 1)
    is_max = logits >= m
    first = jnp.min(jnp.where(is_max, iota, _E), axis=1, keepdims=True)
    masked = jnp.where(iota == first, 1.0 / s, 0.0)

    @pl.when(i == 0)
    def _init():
        colsum_ref[...] = jnp.zeros_like(colsum_ref)

    colsum_ref[...] += jnp.sum(masked, axis=0, keepdims=True)
    out_ref[pl.ds(i * tile, tile), :] = masked

    @pl.when(i == n_tiles - 1)
    def _normalize():
        scale = capacity / (colsum_ref[...] + _EPS)
        out_ref[...] = out_ref[...] * scale


def kernel(x, W, b):
    n_tiles = _N_TOKENS // _TILE
    b2 = b.reshape(1, _E)
    capacity = float(_N_TOKENS)
    return pl.pallas_call(
        partial(_gate_kernel, n_tiles=n_tiles, tile=_TILE, capacity=capacity),
        grid=(n_tiles,),
        in_specs=[
            pl.BlockSpec((_TILE, _DIM), lambda i: (i, 0)),
            pl.BlockSpec((_E, _DIM), lambda i: (0, 0)),
            pl.BlockSpec((1, _E), lambda i: (0, 0)),
        ],
        out_specs=pl.BlockSpec((_N_TOKENS, _E), lambda i: (0, 0)),
        out_shape=jax.ShapeDtypeStruct((_N_TOKENS, _E), jnp.float32),
        scratch_shapes=[pltpu.VMEM((1, _E), jnp.float32)],
    )(x, W, b2)
